# Initial kernel scaffold; baseline (speedup 1.0000x reference)
import jax
import jax.numpy as jnp
from jax.experimental import pallas as pl


def kernel(x_upd_core, edge_index_core, edge_attr_core, Z_core, Z_block, node2graph_core, W_in, b_in, Wm0, bm0, Wu0, bu0, Wm1, bm1, Wu1, bu1, W1, b1, W2, b2):
    raise NotImplementedError("write your pallas kernel here")


# speedup vs baseline: 1.0115x; 1.0115x over previous
import jax
import jax.numpy as jnp
from jax.experimental import pallas as pl
from jax.experimental.pallas import tpu as pltpu

_N = 50000
_NP = 50048
_G = 64


_NR = _NP // 128


def _tail_body(x_ref, w1_ref, b1_ref, w2_ref, b2_ref, ng_ref, p_ref):
    x = x_ref[...]
    h = jnp.maximum(
        jnp.dot(x, w1_ref[...], preferred_element_type=jnp.float32) + b1_ref[0, :], 0.0)
    logit = jnp.dot(h, w2_ref[...], preferred_element_type=jnp.float32) + b2_ref[0, 0]
    lg = jnp.reshape(logit, (_NR, 128))
    nid = (jax.lax.broadcasted_iota(jnp.int32, (_NR, 128), 0) * 128
           + jax.lax.broadcasted_iota(jnp.int32, (_NR, 128), 1))
    lg = jnp.where(nid < _N, lg, -1e30)
    ng = ng_ref[...]
    mxn = jnp.full((_NR, 128), -1e30, dtype=jnp.float32)
    for g in range(_G):
        mask = ng == g
        mx_g = jnp.max(jnp.where(mask, lg, -1e30))
        mxn = jnp.where(mask, mx_g, mxn)
    z = jnp.exp(lg - mxn)
    sn = jnp.ones((_NR, 128), dtype=jnp.float32)
    for g in range(_G):
        mask = ng == g
        s_g = jnp.sum(jnp.where(mask, z, 0.0))
        sn = jnp.where(mask, s_g, sn)
    p_ref[...] = z / sn


def kernel(x_upd_core, edge_index_core, edge_attr_core, Z_core, Z_block,
           node2graph_core, W_in, b_in, Wm0, bm0, Wu0, bu0, Wm1, bm1,
           Wu1, bu1, W1, b1, W2, b2):
    Z_cat = jnp.concatenate([Z_core, Z_block], axis=-1)
    g_node = jnp.take(Z_cat, node2graph_core, axis=0)

    x = jax.nn.relu(x_upd_core @ W_in + b_in)
    src = edge_index_core[0]
    dst = edge_index_core[1]

    for (Wm, bm, Wu, bu) in ((Wm0, bm0, Wu0, bu0), (Wm1, bm1, Wu1, bu1)):
        x_src = jnp.take(x, src, axis=0)
        x_dst = jnp.take(x, dst, axis=0)
        m = jax.nn.relu(jnp.concatenate([x_src, x_dst, edge_attr_core], axis=-1) @ Wm + bm)
        agg = jax.ops.segment_sum(m, dst, num_segments=x.shape[0])
        x = jax.nn.relu(x + jnp.concatenate([x, agg, g_node], axis=-1) @ Wu + bu)

    xp = jnp.pad(x, ((0, _NP - _N), (0, 0)))
    ngp = jnp.pad(node2graph_core, (0, _NP - _N)).reshape(_NR, 128)
    p = pl.pallas_call(
        _tail_body,
        out_shape=jax.ShapeDtypeStruct((_NR, 128), jnp.float32),
    )(xp, W1, b1.reshape(1, -1), W2, b2.reshape(1, 1), ngp)
    return p.reshape(_NP)[:_N]


# speedup vs baseline: 2.2702x; 2.2444x over previous
import functools

import jax
import jax.numpy as jnp
from jax import lax
from jax.experimental import pallas as pl
from jax.experimental.pallas import tpu as pltpu
from jax.experimental.pallas import tpu_sc as plsc

_N = 50000
_NP = 50048
_NR = _NP // 128
_G = 64
_E = 800000

_EPT = _E // 16
_K = 128
_NFULL = _EPT // _K
_KT = _EPT - _NFULL * _K
_HALF = 25088
_NP2 = 2 * _HALF
_GB = 128
_ACC = _HALF + _GB
_ZCH = _ACC // 128
_OCH = _HALF // 128


def _edge_chunk(a_h, b_h, c_h, src_h, dst_h, acc, sidx, didx, lidx, bA, bB, bC,
                s0, s1, s2, base, lo, i, s, n):
    if n == _K:
        sidx_w, didx_w, bA_w, bB_w, bC_w = sidx, didx, bA, bB, bC
    else:
        sidx_w, didx_w = sidx.at[pl.ds(0, n)], didx.at[pl.ds(0, n)]
        bA_w, bB_w, bC_w = (bA.at[pl.ds(0, n)], bB.at[pl.ds(0, n)],
                            bC.at[pl.ds(0, n)])
    pltpu.sync_copy(src_h.at[pl.ds(base, n)], sidx_w)
    pltpu.sync_copy(dst_h.at[pl.ds(base, n)], didx_w)
    cpA = pltpu.async_copy(a_h.at[sidx_w], bA_w, s0)
    cpB = pltpu.async_copy(b_h.at[didx_w], bB_w, s1)
    cpC = pltpu.async_copy(c_h.at[pl.ds(base, n)], bC_w, s2)
    hi = lo + _HALF
    lane = lax.iota(jnp.int32, 16)
    for j in range(n // 16):
        d = didx[pl.ds(j * 16, 16)]
        inh = (d >= lo) & (d < hi)
        goff = (((i + j + s * 4) * 16) & (_GB - 16)) + lane
        lidx[pl.ds(j * 16, 16)] = jnp.where(inh, d - lo, _HALF + goff)
    for j in range(n // 16, _K // 16):
        goff = (((i + j + s * 4) * 16) & (_GB - 16)) + lane
        lidx[pl.ds(j * 16, 16)] = _HALF + goff
    cpA.wait()
    cpB.wait()
    cpC.wait()
    def body(r, _):
        for q in range(4):
            col = q * 16
            m = jnp.maximum(
                bA[r, pl.ds(col, 16)] + bB[r, pl.ds(col, 16)] + bC[r, pl.ds(col, 16)],
                0.0)
            bA[r, pl.ds(col, 16)] = m
        return 0
    lax.fori_loop(0, n, body, 0, unroll=2)
    pltpu.sync_copy(bA, acc.at[lidx], add=True)


def _edge_body(a_h, b_h, c_h, src_h, dst_h, out_h,
               sidx, didx, lidx, bA, bB, bC,
               acc, s0, s1, s2):
    c = lax.axis_index("c")
    s = lax.axis_index("s")
    lo = c * _HALF

    def zrow(r, _):
        for q in range(4):
            bA[r, pl.ds(q * 16, 16)] = jnp.zeros((16,), jnp.float32)
        return 0
    lax.fori_loop(0, _K, zrow, 0, unroll=2)

    def zchunk(i, _):
        j = s + i * 16
        pltpu.sync_copy(bA, acc.at[pl.ds(j * 128, 128)])
        return 0
    lax.fori_loop(0, (_ZCH - s + 15) // 16, zchunk, 0)
    plsc.subcore_barrier()

    ebase = s * _EPT

    def chunk(i, _):
        _edge_chunk(a_h, b_h, c_h, src_h, dst_h, acc,
                    sidx, didx, lidx, bA, bB, bC, s0, s1, s2,
                    ebase + i * _K, lo, i, s, _K)
        return 0
    lax.fori_loop(0, _NFULL, chunk, 0)
    _edge_chunk(a_h, b_h, c_h, src_h, dst_h, acc,
                sidx, didx, lidx, bA, bB, bC, s0, s1, s2,
                ebase + _NFULL * _K, lo, _NFULL, s, _KT)
    plsc.subcore_barrier()

    def ochunk(i, _):
        j = s + i * 16
        pltpu.sync_copy(acc.at[pl.ds(j * 128, 128)],
                        out_h.at[pl.ds(lo + j * 128, 128)])
        return 0
    lax.fori_loop(0, (_OCH - s + 15) // 16, ochunk, 0)


@jax.jit
def _edge_call(A, B, C, src, dst):
    mesh = plsc.VectorSubcoreMesh(core_axis_name="c", subcore_axis_name="s")
    f = pl.kernel(
        _edge_body,
        out_type=jax.ShapeDtypeStruct((_NP2, 64), jnp.float32),
        mesh=mesh,
        scratch_types=[
            pltpu.VMEM((_K,), jnp.int32),
            pltpu.VMEM((_K,), jnp.int32),
            pltpu.VMEM((_K,), jnp.int32),
            pltpu.VMEM((_K, 64), jnp.float32),
            pltpu.VMEM((_K, 64), jnp.float32),
            pltpu.VMEM((_K, 64), jnp.float32),
            pltpu.VMEM_SHARED((_ACC, 64), jnp.float32),
            pltpu.SemaphoreType.DMA,
            pltpu.SemaphoreType.DMA,
            pltpu.SemaphoreType.DMA,
        ],
        compiler_params=pltpu.CompilerParams(use_tc_tiling_on_sc=False),
    )
    return f(A, B, C, src, dst)


def _tail_body(x_ref, w1_ref, b1_ref, w2_ref, b2_ref, ng_ref, p_ref):
    x = x_ref[...]
    h = jnp.maximum(
        jnp.dot(x, w1_ref[...], preferred_element_type=jnp.float32) + b1_ref[0, :], 0.0)
    logit = jnp.dot(h, w2_ref[...], preferred_element_type=jnp.float32) + b2_ref[0, 0]
    lg = jnp.reshape(logit, (_NR, 128))
    nid = (jax.lax.broadcasted_iota(jnp.int32, (_NR, 128), 0) * 128
           + jax.lax.broadcasted_iota(jnp.int32, (_NR, 128), 1))
    lg = jnp.where(nid < _N, lg, -1e30)
    ng = ng_ref[...]
    mxn = jnp.full((_NR, 128), -1e30, dtype=jnp.float32)
    for g in range(_G):
        mask = ng == g
        mx_g = jnp.max(jnp.where(mask, lg, -1e30))
        mxn = jnp.where(mask, mx_g, mxn)
    z = jnp.exp(lg - mxn)
    sn = jnp.ones((_NR, 128), dtype=jnp.float32)
    for g in range(_G):
        mask = ng == g
        s_g = jnp.sum(jnp.where(mask, z, 0.0))
        sn = jnp.where(mask, s_g, sn)
    p_ref[...] = z / sn


def kernel(x_upd_core, edge_index_core, edge_attr_core, Z_core, Z_block,
           node2graph_core, W_in, b_in, Wm0, bm0, Wu0, bu0, Wm1, bm1,
           Wu1, bu1, W1, b1, W2, b2):
    H = 64
    Z_cat = jnp.concatenate([Z_core, Z_block], axis=-1)
    g_node = jnp.take(Z_cat, node2graph_core, axis=0)

    x = jax.nn.relu(x_upd_core @ W_in + b_in)
    src = edge_index_core[0]
    dst = edge_index_core[1]

    for (Wm, bm, Wu, bu) in ((Wm0, bm0, Wu0, bu0), (Wm1, bm1, Wu1, bu1)):
        A = x @ Wm[:H]
        B = x @ Wm[H:2 * H]
        C = edge_attr_core @ Wm[2 * H:] + bm
        agg = _edge_call(A, B, C, src, dst)[:_N]
        x = jax.nn.relu(x + jnp.concatenate([x, agg, g_node], axis=-1) @ Wu + bu)

    xp = jnp.pad(x, ((0, _NP - _N), (0, 0)))
    ngp = jnp.pad(node2graph_core, (0, _NP - _N)).reshape(_NR, 128)
    p = pl.pallas_call(
        _tail_body,
        out_shape=jax.ShapeDtypeStruct((_NR, 128), jnp.float32),
    )(xp, W1, b1.reshape(1, -1), W2, b2.reshape(1, 1), ngp)
    return p.reshape(_NP)[:_N]


# speedup vs baseline: 2.8181x; 1.2413x over previous
import jax
import jax.numpy as jnp
from jax import lax
from jax.experimental import pallas as pl
from jax.experimental.pallas import tpu as pltpu
from jax.experimental.pallas import tpu_sc as plsc

_N = 50000
_G = 64
_E = 800000
_E2 = _E // 2

_EPT = _E // 16
_K = 80
_KH = _K // 2
_HALF = 25088
_NP2 = 2 * _HALF
_NR2 = _NP2 // 128
_ACC = _HALF
_ZCH = _ACC // 64
_OCH = _HALF // 128
_NCH = _EPT // _K
_PB = _N
_RCH = 3136


def _valu_idx(didx, lidx, lo, i, s):
    hi = lo + _HALF
    lane = lax.iota(jnp.int32, 16)
    for j in range(_K // 16):
        d = didx[pl.ds(j * 16, 16)]
        inh = (d >= lo) & (d < hi)
        poff = (((i + j + s * 4) * 16) & 112) + lane
        didx[pl.ds(j * 16, 16)] = jnp.where(inh, d, _PB + poff)
        lidx[pl.ds(j * 16, 16)] = jnp.where(d >= _HALF, d - _HALF, d)


def _relu_rows(bA, bB, bC):
    def body(rp, _):
        for half in range(2):
            r = rp * 2 + half
            for q in range(4):
                col = q * 16
                m = jnp.maximum(
                    bA[r, pl.ds(col, 16)] + bB[r, pl.ds(col, 16)]
                    + bC[rp, pl.ds(half * 64 + col, 16)], 0.0)
                bA[r, pl.ds(col, 16)] = m
        return 0
    lax.fori_loop(0, _KH, body, 0, unroll=2)


def _edge_body(a_h, b_h, c_h, src_h, dst_h, out_h,
               sidx0, didx0, lidx0, bA0, bB0,
               sidx1, didx1, lidx1, bA1, bB1,
               bC, acc, sI0, sI1, sG0, sG1, sS0, sS1, sC):
    c = lax.axis_index("c")
    s = lax.axis_index("s")
    lo = c * _HALF
    sidx = (sidx0, sidx1)
    didx = (didx0, didx1)
    lidx = (lidx0, lidx1)
    bA = (bA0, bA1)
    bB = (bB0, bB1)
    sI = (sI0, sI1)
    sG = (sG0, sG1)
    sS = (sS0, sS1)

    def zrow(r, _):
        for q in range(4):
            bA0[r, pl.ds(q * 16, 16)] = jnp.zeros((16,), jnp.float32)
        return 0
    lax.fori_loop(0, 64, zrow, 0, unroll=4)

    def zchunk(i, _):
        j = s + i * 16
        pltpu.sync_copy(bA0.at[pl.ds(0, 64)], acc.at[pl.ds(j * 64, 64)])
        return 0
    lax.fori_loop(0, (_ZCH - s + 15) // 16, zchunk, 0)
    plsc.subcore_barrier()

    ebase = s * _EPT
    cbase = s * (_EPT // 2)

    def issue_idx(i, b):
        pltpu.async_copy(src_h.at[pl.ds(ebase + i * _K, _K)], sidx[b], sI[b])
        pltpu.async_copy(dst_h.at[pl.ds(ebase + i * _K, _K)], didx[b], sI[b])

    def wait_idx(b):
        pltpu.make_async_copy(src_h.at[pl.ds(0, _K)], sidx[b], sI[b]).wait()
        pltpu.make_async_copy(dst_h.at[pl.ds(0, _K)], didx[b], sI[b]).wait()

    def issue_gathers(b):
        pltpu.async_copy(a_h.at[sidx[b]], bA[b], sG[b])
        pltpu.async_copy(b_h.at[didx[b]], bB[b], sG[b])

    def wait_gathers(b):
        pltpu.make_async_copy(a_h.at[sidx[b]], bA[b], sG[b]).wait()
        pltpu.make_async_copy(b_h.at[didx[b]], bB[b], sG[b]).wait()

    def issue_scatter(b):
        pltpu.async_copy(bA[b], acc.at[lidx[b]], sS[b], add=True)

    def wait_scatter(b):
        pltpu.make_async_copy(bA[b], acc.at[lidx[b]], sS[b]).wait()

    issue_idx(0, 0)
    wait_idx(0)
    _valu_idx(didx[0], lidx[0], lo, 0, s)
    issue_gathers(0)
    issue_idx(1, 1)

    def body(i, _):
        b = lax.rem(i, 2)
        pltpu.async_copy(c_h.at[pl.ds(cbase + i * _KH, _KH)], bC, sC)
        for bb in range(2):

            @pl.when(b == bb)
            def _():
                bo = 1 - bb

                @pl.when(i >= 1)
                def _():
                    wait_scatter(bo)

                @pl.when(i + 1 < _NCH)
                def _():
                    wait_idx(bo)
                    _valu_idx(didx[bo], lidx[bo], lo, i + 1, s)
                    issue_gathers(bo)
                pltpu.make_async_copy(c_h.at[pl.ds(0, _KH)], bC, sC).wait()
                wait_gathers(bb)

                @pl.when(i + 2 < _NCH)
                def _():
                    issue_idx(i + 2, bb)
                _relu_rows(bA[bb], bB[bb], bC)
                issue_scatter(bb)
        return 0
    lax.fori_loop(0, _NCH, body, 0)
    wait_scatter((_NCH - 1) % 2)
    plsc.subcore_barrier()

    def ochunk(i, _):
        j = s + i * 16
        pltpu.sync_copy(acc.at[pl.ds(j * 128, 128)],
                        out_h.at[pl.ds(lo + j * 128, 128)])
        return 0
    lax.fori_loop(0, (_OCH - s + 15) // 16, ochunk, 0)


@jax.jit
def _edge_call(A, B, C, src, dst):
    mesh = plsc.VectorSubcoreMesh(core_axis_name="c", subcore_axis_name="s")
    f = pl.kernel(
        _edge_body,
        out_type=jax.ShapeDtypeStruct((_NP2, 64), jnp.float32),
        mesh=mesh,
        scratch_types=(
            [pltpu.VMEM((_K,), jnp.int32)] * 3
            + [pltpu.VMEM((_K, 64), jnp.float32)] * 2
            + [pltpu.VMEM((_K,), jnp.int32)] * 3
            + [pltpu.VMEM((_K, 64), jnp.float32)] * 2
            + [pltpu.VMEM((_KH, 128), jnp.float32)]
            + [pltpu.VMEM_SHARED((_ACC, 64), jnp.float32)]
            + [pltpu.SemaphoreType.DMA] * 7
        ),
        compiler_params=pltpu.CompilerParams(use_tc_tiling_on_sc=False),
    )
    return f(A, B, C, src, dst)


def _row_poison(vals, i):
    nid = (jax.lax.broadcasted_iota(jnp.int32, (_RCH, 64), 0)
           + i * _RCH)
    return jnp.where(nid < _N, vals, -1e30)


def _prep_body(xu_ref, win_ref, bin_ref, wab_ref, x_ref, a_ref, b_ref):
    i = pl.program_id(0)
    x = jnp.maximum(
        jnp.dot(xu_ref[...], win_ref[...], preferred_element_type=jnp.float32)
        + bin_ref[0, :], 0.0)
    x_ref[...] = x
    a_ref[...] = jnp.dot(x, wab_ref[...][:64], preferred_element_type=jnp.float32)
    b_ref[...] = _row_poison(
        jnp.dot(x, wab_ref[...][64:], preferred_element_type=jnp.float32), i)


def _upd_body(x_ref, agg_ref, oh_ref, zc_ref, wu_ref, bu_ref, wab_ref,
              xn_ref, an_ref, bn_ref):
    i = pl.program_id(0)
    wu = wu_ref[...]
    gu = jnp.dot(zc_ref[...], wu[128:], preferred_element_type=jnp.float32)
    gn = jnp.dot(oh_ref[...], gu, preferred_element_type=jnp.float32)
    x = x_ref[...]
    xn = jnp.maximum(
        x + jnp.dot(x, wu[:64], preferred_element_type=jnp.float32)
        + jnp.dot(agg_ref[...], wu[64:128], preferred_element_type=jnp.float32)
        + gn + bu_ref[0, :], 0.0)
    xn_ref[...] = xn
    an_ref[...] = jnp.dot(xn, wab_ref[...][:64], preferred_element_type=jnp.float32)
    bn_ref[...] = _row_poison(
        jnp.dot(xn, wab_ref[...][64:], preferred_element_type=jnp.float32), i)


def _cmat_body(attr2_ref, w_ref, b_ref, c0_ref, c1_ref):
    y = (jnp.dot(attr2_ref[...], w_ref[...], preferred_element_type=jnp.float32)
         + b_ref[0, :])
    c0_ref[...] = y[:, :128]
    c1_ref[...] = y[:, 128:]


def _tail_body(x_ref, w1_ref, b1_ref, w2_ref, b2_ref, ng_ref, p_ref):
    x = x_ref[...]
    h = jnp.maximum(
        jnp.dot(x, w1_ref[...], preferred_element_type=jnp.float32) + b1_ref[0, :], 0.0)
    logit = jnp.dot(h, w2_ref[...], preferred_element_type=jnp.float32) + b2_ref[0, 0]
    lg = jnp.reshape(logit, (_NR2, 128))
    nid = (jax.lax.broadcasted_iota(jnp.int32, (_NR2, 128), 0) * 128
           + jax.lax.broadcasted_iota(jnp.int32, (_NR2, 128), 1))
    lg = jnp.where(nid < _N, lg, -1e30)
    ng = ng_ref[...]
    mxn = jnp.full((_NR2, 128), -1e30, dtype=jnp.float32)
    for g in range(_G):
        mask = ng == g
        mx_g = jnp.max(jnp.where(mask, lg, -1e30))
        mxn = jnp.where(mask, mx_g, mxn)
    z = jnp.exp(lg - mxn)
    sn = jnp.ones((_NR2, 128), dtype=jnp.float32)
    for g in range(_G):
        mask = ng == g
        s_g = jnp.sum(jnp.where(mask, z, 0.0))
        sn = jnp.where(mask, s_g, sn)
    p_ref[...] = z / sn


_ROWSPEC = pl.BlockSpec((_RCH, 64), lambda i: (i, 0))
_NGSPEC = _ROWSPEC


def _full(shape):
    return pl.BlockSpec(shape, lambda i: tuple(0 for _ in shape))


def kernel(x_upd_core, edge_index_core, edge_attr_core, Z_core, Z_block,
           node2graph_core, W_in, b_in, Wm0, bm0, Wu0, bu0, Wm1, bm1,
           Wu1, bu1, W1, b1, W2, b2):
    H = 64
    f32 = jnp.float32
    Z_cat = jnp.concatenate([Z_core, Z_block], axis=-1)
    src = edge_index_core[0]
    dst = edge_index_core[1]
    ng2 = jnp.pad(node2graph_core, (0, _NP2 - _N)).reshape(_NR2, 128)
    oh = (jnp.pad(node2graph_core, (0, _NP2 - _N))[:, None]
          == jnp.arange(_G, dtype=jnp.int32)[None, :]).astype(f32)
    xu_p = jnp.pad(x_upd_core, ((0, _NP2 - _N), (0, 0)))
    attr2 = edge_attr_core.reshape(_E2, 32)

    Wc = jnp.zeros((32, 256), f32)
    Wc = Wc.at[:16, 0:64].set(Wm0[2 * H:])
    Wc = Wc.at[16:, 64:128].set(Wm0[2 * H:])
    Wc = Wc.at[:16, 128:192].set(Wm1[2 * H:])
    Wc = Wc.at[16:, 192:256].set(Wm1[2 * H:])
    bc = jnp.concatenate([bm0, bm0, bm1, bm1]).reshape(1, 256)

    C0, C1 = pl.pallas_call(
        _cmat_body,
        grid=(100,),
        in_specs=[pl.BlockSpec((4000, 32), lambda i: (i, 0)),
                  _full((32, 256)), _full((1, 256))],
        out_specs=[pl.BlockSpec((4000, 128), lambda i: (i, 0)),
                   pl.BlockSpec((4000, 128), lambda i: (i, 0))],
        out_shape=[jax.ShapeDtypeStruct((_E2, 128), f32),
                   jax.ShapeDtypeStruct((_E2, 128), f32)],
    )(attr2, Wc, bc)

    x0, A0, B0 = pl.pallas_call(
        _prep_body,
        grid=(16,),
        in_specs=[_ROWSPEC, _full((64, 64)), _full((1, 64)),
                  _full((128, 64))],
        out_specs=[_ROWSPEC, _ROWSPEC, _ROWSPEC],
        out_shape=[jax.ShapeDtypeStruct((_NP2, 64), f32)] * 3,
    )(xu_p, W_in, b_in.reshape(1, -1), Wm0[:2 * H])

    def upd(x, agg, Wu, bu, Wm_next):
        return pl.pallas_call(
            _upd_body,
            grid=(16,),
            in_specs=[_ROWSPEC, _ROWSPEC, _NGSPEC, _full((64, 128)),
                      _full((256, 64)), _full((1, 64)), _full((128, 64))],
            out_specs=[_ROWSPEC, _ROWSPEC, _ROWSPEC],
            out_shape=[jax.ShapeDtypeStruct((_NP2, 64), f32)] * 3,
        )(x, agg, oh, Z_cat, Wu, bu.reshape(1, -1), Wm_next)

    agg0 = _edge_call(A0, B0, C0, src, dst)
    x1, A1, B1 = upd(x0, agg0, Wu0, bu0, Wm1[:2 * H])
    agg1 = _edge_call(A1, B1, C1, src, dst)
    x2, _, _ = upd(x1, agg1, Wu1, bu1, Wm1[:2 * H])

    p = pl.pallas_call(
        _tail_body,
        out_shape=jax.ShapeDtypeStruct((_NR2, 128), f32),
    )(x2, W1, b1.reshape(1, -1), W2, b2.reshape(1, 1), ng2)
    return p.reshape(_NP2)[:_N]
